# baseline (speedup 1.0000x reference)
import functools

import numpy as np
import jax
import jax.numpy as jnp
from jax import lax
from jax.experimental import pallas as pl
from jax.experimental.pallas import tpu as pltpu
from jax.experimental.pallas import tpu_sc as plsc

T, D, E, K, F = 2048, 1024, 8, 2, 512
BT = 256
PT = T * K + E * BT
NB = PT // BT
NMETA = 32
BTC = 256
NC, NS = 2, 16
NW = NC * NS
CHUNK = T // NW
DW = D // 2

_MTRI_NP = np.triu(np.ones((T, T), np.float32), 1)


def _router_body(x_ref, Wr_ref, mtri_ref, slots_ref, meta_ref, loss_ref):
    x = x_ref[...]
    logits_t = lax.dot_general(Wr_ref[...], x, (((0,), (1,)), ((), ())),
                               preferred_element_type=jnp.float32)
    m = jnp.max(logits_t, axis=0, keepdims=True)
    ex = jnp.exp(logits_t - m)
    p = ex / jnp.sum(ex, axis=0, keepdims=True)
    eio = lax.broadcasted_iota(jnp.int32, (E, T), 0)
    m1 = jnp.max(p, axis=0, keepdims=True)
    i1 = jnp.min(jnp.where(p == m1, eio, E), axis=0, keepdims=True)
    pm = jnp.where(eio == i1, -jnp.inf, p)
    m2 = jnp.max(pm, axis=0, keepdims=True)
    i2 = jnp.min(jnp.where(pm == m2, eio, E), axis=0, keepdims=True)
    sel = ((eio == i1) | (eio == i2)).astype(jnp.bfloat16)

    rank = lax.dot_general(sel, mtri_ref[...], (((1,), (0,)), ((), ())),
                           preferred_element_type=jnp.float32)

    counts = jnp.sum(sel.astype(jnp.float32), axis=1, keepdims=True)
    padded = jnp.ceil(counts / BT) * BT
    eio_r = lax.broadcasted_iota(jnp.int32, (E, E), 0)
    eio_c = lax.broadcasted_iota(jnp.int32, (E, E), 1)
    metri = (eio_c < eio_r).astype(jnp.float32)
    starts = lax.dot_general(metri, padded, (((1,), (0,)), ((), ())),
                             preferred_element_type=jnp.float32)
    slotmat = starts + rank
    slot0 = jnp.sum(jnp.where(eio == i1, slotmat, 0.0), axis=0, keepdims=True)
    slot1 = jnp.sum(jnp.where(eio == i2, slotmat, 0.0), axis=0, keepdims=True)
    slots_ref[...] = jnp.concatenate([slot0, slot1], axis=0).astype(jnp.int32)

    ends = starts + padded
    bio = lax.broadcasted_iota(jnp.int32, (1, NMETA), 1).astype(jnp.float32) * BT
    bexp = jnp.sum((bio >= ends).astype(jnp.int32), axis=0, keepdims=True)
    bexp = jnp.minimum(bexp, E - 1)
    total = jnp.sum(padded, axis=0, keepdims=True)
    bvalid = (bio < total).astype(jnp.int32)
    meta_ref[...] = jnp.concatenate([bexp, bvalid], axis=0)

    psum = jnp.sum(p, axis=1, keepdims=True)
    loss = E * jnp.sum((counts / (T * K)) * (psum / T), keepdims=True)
    loss_ref[...] = loss


def _router(x, Wr, mtri):
    return pl.pallas_call(
        _router_body,
        in_specs=[
            pl.BlockSpec((T, D), lambda: (0, 0)),
            pl.BlockSpec((D, E), lambda: (0, 0)),
            pl.BlockSpec((T, T), lambda: (0, 0)),
        ],
        out_specs=[
            pl.BlockSpec((2, T), lambda: (0, 0)),
            pl.BlockSpec((2, NMETA), lambda: (0, 0)),
            pl.BlockSpec((1, 1), lambda: (0, 0)),
        ],
        out_shape=[
            jax.ShapeDtypeStruct((2, T), jnp.int32),
            jax.ShapeDtypeStruct((2, NMETA), jnp.int32),
            jax.ShapeDtypeStruct((1, 1), jnp.float32),
        ],
    )(x, Wr, mtri)


_sc_mesh = plsc.VectorSubcoreMesh(core_axis_name="c", subcore_axis_name="s",
                                  num_cores=NC, num_subcores=NS)


@functools.partial(
    pl.kernel,
    out_type=jax.ShapeDtypeStruct((PT, DW), jnp.int32),
    mesh=_sc_mesh,
    scratch_types=[
        pltpu.VMEM((CHUNK,), jnp.int32),
        pltpu.VMEM((CHUNK,), jnp.int32),
        pltpu.VMEM((CHUNK, DW), jnp.int32),
        pltpu.SemaphoreType.DMA,
        pltpu.SemaphoreType.DMA,
    ],
)
def _dispatch(x_hbm, slots_hbm, xs_hbm, idx0_v, idx1_v, rows_v, sem0, sem1):
    wid = lax.axis_index("s") * NC + lax.axis_index("c")
    base = wid * CHUNK
    pltpu.sync_copy(slots_hbm.at[0, pl.ds(base, CHUNK)], idx0_v)
    pltpu.sync_copy(slots_hbm.at[1, pl.ds(base, CHUNK)], idx1_v)
    pltpu.sync_copy(x_hbm.at[pl.ds(base, CHUNK)], rows_v)
    c0 = pltpu.async_copy(rows_v, xs_hbm.at[idx0_v], sem0)
    c1 = pltpu.async_copy(rows_v, xs_hbm.at[idx1_v], sem1)
    c0.wait()
    c1.wait()


def _gmm_body(meta_ref, xs_ref, W0_ref, W1_ref, Wo_ref, ys_ref,
              w0_s, w1_s, wo_s):
    b = pl.program_id(0)

    new_exp = (b == 0) | (meta_ref[0, b] != meta_ref[0, jnp.maximum(b - 1, 0)])

    @pl.when(new_exp)
    def _cast_weights():
        w0_s[...] = W0_ref[0].astype(jnp.bfloat16)
        w1_s[...] = W1_ref[0].astype(jnp.bfloat16)
        wo_s[...] = Wo_ref[0].astype(jnp.bfloat16)

    @pl.when(meta_ref[1, b] == 1)
    def _():
        xb = xs_ref[...]
        h0 = jnp.dot(xb, w0_s[...], preferred_element_type=jnp.float32)
        h1 = jnp.dot(xb, w1_s[...], preferred_element_type=jnp.float32)
        h = (jax.nn.silu(h0) * h1).astype(jnp.bfloat16)
        ys_ref[...] = jnp.dot(h, wo_s[...],
                              preferred_element_type=jnp.float32
                              ).astype(jnp.bfloat16)


def _gmm(meta, xs, W0, W1, Wo):
    grid_spec = pltpu.PrefetchScalarGridSpec(
        num_scalar_prefetch=1,
        grid=(NB,),
        in_specs=[
            pl.BlockSpec((BT, D), lambda b, meta: (b, 0)),
            pl.BlockSpec((1, D, F), lambda b, meta: (meta[0, b], 0, 0)),
            pl.BlockSpec((1, D, F), lambda b, meta: (meta[0, b], 0, 0)),
            pl.BlockSpec((1, F, D), lambda b, meta: (meta[0, b], 0, 0)),
        ],
        out_specs=pl.BlockSpec((BT, D), lambda b, meta: (b, 0)),
        scratch_shapes=[
            pltpu.VMEM((D, F), jnp.bfloat16),
            pltpu.VMEM((D, F), jnp.bfloat16),
            pltpu.VMEM((F, D), jnp.bfloat16),
        ],
    )
    return pl.pallas_call(
        _gmm_body,
        grid_spec=grid_spec,
        out_shape=jax.ShapeDtypeStruct((PT, D), jnp.bfloat16),
        compiler_params=pltpu.CompilerParams(
            dimension_semantics=("arbitrary",),
        ),
    )(meta, xs, W0, W1, Wo)


@functools.partial(
    pl.kernel,
    out_type=[
        jax.ShapeDtypeStruct((T, DW), jnp.int32),
        jax.ShapeDtypeStruct((T, DW), jnp.int32),
    ],
    mesh=_sc_mesh,
    scratch_types=[
        pltpu.VMEM((CHUNK,), jnp.int32),
        pltpu.VMEM((CHUNK,), jnp.int32),
        pltpu.VMEM((CHUNK, DW), jnp.int32),
        pltpu.VMEM((CHUNK, DW), jnp.int32),
        pltpu.SemaphoreType.DMA,
        pltpu.SemaphoreType.DMA,
    ],
)
def _undispatch(ys_hbm, slots_hbm, op0_hbm, op1_hbm, idx0_v, idx1_v,
                rows0_v, rows1_v, sem0, sem1):
    wid = lax.axis_index("s") * NC + lax.axis_index("c")
    base = wid * CHUNK
    pltpu.sync_copy(slots_hbm.at[0, pl.ds(base, CHUNK)], idx0_v)
    pltpu.sync_copy(slots_hbm.at[1, pl.ds(base, CHUNK)], idx1_v)
    c0 = pltpu.async_copy(ys_hbm.at[idx0_v], rows0_v, sem0)
    c1 = pltpu.async_copy(ys_hbm.at[idx1_v], rows1_v, sem1)
    c0.wait()
    pltpu.sync_copy(rows0_v, op0_hbm.at[pl.ds(base, CHUNK)])
    c1.wait()
    pltpu.sync_copy(rows1_v, op1_hbm.at[pl.ds(base, CHUNK)])


def _combine_body(x_ref, Wr_ref, op0_ref, op1_ref, Ws0_ref, Ws1_ref,
                  Wso_ref, Wg_ref, out_ref, ws0_s, ws1_s, wso_s):
    t = pl.program_id(0)

    @pl.when(t == 0)
    def _cast_weights():
        ws0_s[...] = Ws0_ref[...].astype(jnp.bfloat16)
        ws1_s[...] = Ws1_ref[...].astype(jnp.bfloat16)
        wso_s[...] = Wso_ref[...].astype(jnp.bfloat16)

    x = x_ref[...]
    logits = jnp.dot(x, Wr_ref[...], preferred_element_type=jnp.float32)
    m = jnp.max(logits, axis=1, keepdims=True)
    ex = jnp.exp(logits - m)
    p = ex / jnp.sum(ex, axis=1, keepdims=True)
    iota = lax.broadcasted_iota(jnp.int32, (BTC, E), 1)
    m1 = jnp.max(p, axis=1, keepdims=True)
    i1 = jnp.min(jnp.where(p == m1, iota, E), axis=1, keepdims=True)
    pm = jnp.where(iota == i1, -jnp.inf, p)
    m2 = jnp.max(pm, axis=1, keepdims=True)
    denom = m1 + m2
    w0 = m1 / denom
    w1 = m2 / denom
    xb = x.astype(jnp.bfloat16)
    h0s = jnp.dot(xb, ws0_s[...], preferred_element_type=jnp.float32)
    h1s = jnp.dot(xb, ws1_s[...], preferred_element_type=jnp.float32)
    hs = (jax.nn.silu(h0s) * h1s).astype(jnp.bfloat16)
    sh = jnp.dot(hs, wso_s[...], preferred_element_type=jnp.float32)
    g = jax.nn.sigmoid(jnp.dot(x, Wg_ref[...],
                               preferred_element_type=jnp.float32))
    out_ref[...] = (w0 * op0_ref[...].astype(jnp.float32)
                    + w1 * op1_ref[...].astype(jnp.float32) + g * sh)


def _combine(x, Wr, op0, op1, Ws0, Ws1, Wso, Wg):
    nblk = T // BTC
    return pl.pallas_call(
        _combine_body,
        grid=(nblk,),
        in_specs=[
            pl.BlockSpec((BTC, D), lambda t: (t, 0)),
            pl.BlockSpec((D, E), lambda t: (0, 0)),
            pl.BlockSpec((BTC, D), lambda t: (t, 0)),
            pl.BlockSpec((BTC, D), lambda t: (t, 0)),
            pl.BlockSpec((D, F), lambda t: (0, 0)),
            pl.BlockSpec((D, F), lambda t: (0, 0)),
            pl.BlockSpec((F, D), lambda t: (0, 0)),
            pl.BlockSpec((D, 1), lambda t: (0, 0)),
        ],
        out_specs=pl.BlockSpec((BTC, D), lambda t: (t, 0)),
        out_shape=jax.ShapeDtypeStruct((T, D), jnp.float32),
        scratch_shapes=[
            pltpu.VMEM((D, F), jnp.bfloat16),
            pltpu.VMEM((D, F), jnp.bfloat16),
            pltpu.VMEM((F, D), jnp.bfloat16),
        ],
    )(x, Wr, op0, op1, Ws0, Ws1, Wso, Wg)


def _bf16_to_i32(a):
    return lax.bitcast_convert_type(
        a.reshape(a.shape[0], a.shape[1] // 2, 2), jnp.int32)


def _i32_to_bf16(a):
    return lax.bitcast_convert_type(a, jnp.bfloat16).reshape(
        a.shape[0], a.shape[1] * 2)


@jax.jit
def _moe(x, Wr, W0, W1, Wo, Ws0, Ws1, Wso, Wg):
    mtri = jnp.asarray(_MTRI_NP, jnp.bfloat16)
    slots, meta, loss = _router(x, Wr, mtri)
    x_i32 = _bf16_to_i32(x.astype(jnp.bfloat16))
    xs_i32 = _dispatch(x_i32, slots)
    ys = _gmm(meta, _i32_to_bf16(xs_i32), W0, W1, Wo)
    op0_i32, op1_i32 = _undispatch(_bf16_to_i32(ys), slots)
    out = _combine(x, Wr, _i32_to_bf16(op0_i32), _i32_to_bf16(op1_i32),
                   Ws0, Ws1, Wso, Wg)
    return out, loss[0, 0]


def kernel(hidden_states, Wr, W_in0, W_in1, W_out, Ws_in0, Ws_in1, Ws_out,
           Wg, deterministic=True):
    b, s, d = hidden_states.shape
    x = hidden_states.reshape(-1, d)
    out, loss = _moe(x, Wr, W_in0, W_in1, W_out, Ws_in0, Ws_in1, Ws_out, Wg)
    return out.reshape(b, s, d), loss


# speedup vs baseline: 4.1933x; 4.1933x over previous
import functools

import numpy as np
import jax
import jax.numpy as jnp
from jax import lax
from jax.experimental import pallas as pl
from jax.experimental.pallas import tpu as pltpu
from jax.experimental.pallas import tpu_sc as plsc

T, D, E, K, F = 2048, 1024, 8, 2, 512
BT = 256
PT = T * K + E * BT
NB = PT // BT
NMETA = 32
BTC = 256
NC, NS = 2, 16
NW = NC * NS
CHUNK = T // NW
DW = D // 2

_MTRI_NP = np.triu(np.ones((T, T), np.float32), 1)


def _router_body(x_ref, Wr_ref, mtri_ref, slots_ref, meta_ref, loss_ref):
    x = x_ref[...]
    logits_t = lax.dot_general(Wr_ref[...], x, (((0,), (1,)), ((), ())),
                               preferred_element_type=jnp.float32)
    m = jnp.max(logits_t, axis=0, keepdims=True)
    ex = jnp.exp(logits_t - m)
    p = ex / jnp.sum(ex, axis=0, keepdims=True)
    eio = lax.broadcasted_iota(jnp.int32, (E, T), 0)
    m1 = jnp.max(p, axis=0, keepdims=True)
    i1 = jnp.min(jnp.where(p == m1, eio, E), axis=0, keepdims=True)
    pm = jnp.where(eio == i1, -jnp.inf, p)
    m2 = jnp.max(pm, axis=0, keepdims=True)
    i2 = jnp.min(jnp.where(pm == m2, eio, E), axis=0, keepdims=True)
    sel = ((eio == i1) | (eio == i2)).astype(jnp.bfloat16)

    rank = lax.dot_general(sel, mtri_ref[...], (((1,), (0,)), ((), ())),
                           preferred_element_type=jnp.float32)

    counts = jnp.sum(sel.astype(jnp.float32), axis=1, keepdims=True)
    padded = jnp.ceil(counts / BT) * BT
    eio_r = lax.broadcasted_iota(jnp.int32, (E, E), 0)
    eio_c = lax.broadcasted_iota(jnp.int32, (E, E), 1)
    metri = (eio_c < eio_r).astype(jnp.float32)
    starts = lax.dot_general(metri, padded, (((1,), (0,)), ((), ())),
                             preferred_element_type=jnp.float32)
    slotmat = starts + rank
    slot0 = jnp.sum(jnp.where(eio == i1, slotmat, 0.0), axis=0, keepdims=True)
    slot1 = jnp.sum(jnp.where(eio == i2, slotmat, 0.0), axis=0, keepdims=True)
    slots_ref[...] = jnp.concatenate([slot0, slot1], axis=0).astype(jnp.int32)

    ends = starts + padded
    bio = lax.broadcasted_iota(jnp.int32, (1, NMETA), 1).astype(jnp.float32) * BT
    bexp = jnp.sum((bio >= ends).astype(jnp.int32), axis=0, keepdims=True)
    bexp = jnp.minimum(bexp, E - 1)
    total = jnp.sum(padded, axis=0, keepdims=True)
    bvalid = (bio < total).astype(jnp.int32)
    meta_ref[...] = jnp.concatenate([bexp, bvalid], axis=0)

    psum = jnp.sum(p, axis=1, keepdims=True)
    loss = E * jnp.sum((counts / (T * K)) * (psum / T), keepdims=True)
    loss_ref[...] = loss


def _router(x, Wr, mtri):
    return pl.pallas_call(
        _router_body,
        in_specs=[
            pl.BlockSpec((T, D), lambda: (0, 0)),
            pl.BlockSpec((D, E), lambda: (0, 0)),
            pl.BlockSpec((T, T), lambda: (0, 0)),
        ],
        out_specs=[
            pl.BlockSpec((2, T), lambda: (0, 0)),
            pl.BlockSpec((2, NMETA), lambda: (0, 0)),
            pl.BlockSpec((1, 1), lambda: (0, 0)),
        ],
        out_shape=[
            jax.ShapeDtypeStruct((2, T), jnp.int32),
            jax.ShapeDtypeStruct((2, NMETA), jnp.int32),
            jax.ShapeDtypeStruct((1, 1), jnp.float32),
        ],
    )(x, Wr, mtri)


_sc_mesh = plsc.VectorSubcoreMesh(core_axis_name="c", subcore_axis_name="s",
                                  num_cores=NC, num_subcores=NS)


@functools.partial(
    pl.kernel,
    out_type=jax.ShapeDtypeStruct((PT, D), jnp.float32),
    mesh=_sc_mesh,
    scratch_types=[
        pltpu.VMEM((CHUNK,), jnp.int32),
        pltpu.VMEM((CHUNK,), jnp.int32),
        pltpu.VMEM((CHUNK, D), jnp.float32),
        pltpu.SemaphoreType.DMA,
        pltpu.SemaphoreType.DMA,
    ],
)
def _dispatch(x_hbm, slots_hbm, xs_hbm, idx0_v, idx1_v, rows_v, sem0, sem1):
    wid = lax.axis_index("s") * NC + lax.axis_index("c")
    base = wid * CHUNK
    pltpu.sync_copy(slots_hbm.at[0, pl.ds(base, CHUNK)], idx0_v)
    pltpu.sync_copy(slots_hbm.at[1, pl.ds(base, CHUNK)], idx1_v)
    pltpu.sync_copy(x_hbm.at[pl.ds(base, CHUNK)], rows_v)
    c0 = pltpu.async_copy(rows_v, xs_hbm.at[idx0_v], sem0)
    c1 = pltpu.async_copy(rows_v, xs_hbm.at[idx1_v], sem1)
    c0.wait()
    c1.wait()


def _gmm_body(meta_ref, xs_ref, W0_ref, W1_ref, Wo_ref, ys_ref,
              w0_s, w1_s, wo_s):
    b = pl.program_id(0)

    new_exp = (b == 0) | (meta_ref[0, b] != meta_ref[0, jnp.maximum(b - 1, 0)])

    @pl.when(new_exp)
    def _cast_weights():
        w0_s[...] = W0_ref[0].astype(jnp.bfloat16)
        w1_s[...] = W1_ref[0].astype(jnp.bfloat16)
        wo_s[...] = Wo_ref[0].astype(jnp.bfloat16)

    @pl.when(meta_ref[1, b] == 1)
    def _():
        xb = xs_ref[...].astype(jnp.bfloat16)
        h0 = jnp.dot(xb, w0_s[...], preferred_element_type=jnp.float32)
        h1 = jnp.dot(xb, w1_s[...], preferred_element_type=jnp.float32)
        h = (jax.nn.silu(h0) * h1).astype(jnp.bfloat16)
        ys_ref[...] = jnp.dot(h, wo_s[...],
                              preferred_element_type=jnp.float32)


def _gmm(meta, xs, W0, W1, Wo):
    grid_spec = pltpu.PrefetchScalarGridSpec(
        num_scalar_prefetch=1,
        grid=(NB,),
        in_specs=[
            pl.BlockSpec((BT, D), lambda b, meta: (b, 0)),
            pl.BlockSpec((1, D, F), lambda b, meta: (meta[0, b], 0, 0)),
            pl.BlockSpec((1, D, F), lambda b, meta: (meta[0, b], 0, 0)),
            pl.BlockSpec((1, F, D), lambda b, meta: (meta[0, b], 0, 0)),
        ],
        out_specs=pl.BlockSpec((BT, D), lambda b, meta: (b, 0)),
        scratch_shapes=[
            pltpu.VMEM((D, F), jnp.bfloat16),
            pltpu.VMEM((D, F), jnp.bfloat16),
            pltpu.VMEM((F, D), jnp.bfloat16),
        ],
    )
    return pl.pallas_call(
        _gmm_body,
        grid_spec=grid_spec,
        out_shape=jax.ShapeDtypeStruct((PT, D), jnp.float32),
        compiler_params=pltpu.CompilerParams(
            dimension_semantics=("arbitrary",),
        ),
    )(meta, xs, W0, W1, Wo)


@functools.partial(
    pl.kernel,
    out_type=[
        jax.ShapeDtypeStruct((T, D), jnp.float32),
        jax.ShapeDtypeStruct((T, D), jnp.float32),
    ],
    mesh=_sc_mesh,
    scratch_types=[
        pltpu.VMEM((CHUNK,), jnp.int32),
        pltpu.VMEM((CHUNK,), jnp.int32),
        pltpu.VMEM((CHUNK, D), jnp.float32),
        pltpu.SemaphoreType.DMA,
    ],
)
def _undispatch(ys_hbm, slots_hbm, op0_hbm, op1_hbm, idx0_v, idx1_v,
                rows_v, sem):
    wid = lax.axis_index("s") * NC + lax.axis_index("c")
    base = wid * CHUNK
    pltpu.sync_copy(slots_hbm.at[0, pl.ds(base, CHUNK)], idx0_v)
    pltpu.sync_copy(slots_hbm.at[1, pl.ds(base, CHUNK)], idx1_v)
    pltpu.async_copy(ys_hbm.at[idx0_v], rows_v, sem).wait()
    pltpu.sync_copy(rows_v, op0_hbm.at[pl.ds(base, CHUNK)])
    pltpu.async_copy(ys_hbm.at[idx1_v], rows_v, sem).wait()
    pltpu.sync_copy(rows_v, op1_hbm.at[pl.ds(base, CHUNK)])


def _combine_body(x_ref, Wr_ref, op0_ref, op1_ref, Ws0_ref, Ws1_ref,
                  Wso_ref, Wg_ref, out_ref, ws0_s, ws1_s, wso_s):
    t = pl.program_id(0)

    @pl.when(t == 0)
    def _cast_weights():
        ws0_s[...] = Ws0_ref[...].astype(jnp.bfloat16)
        ws1_s[...] = Ws1_ref[...].astype(jnp.bfloat16)
        wso_s[...] = Wso_ref[...].astype(jnp.bfloat16)

    x = x_ref[...]
    logits = jnp.dot(x, Wr_ref[...], preferred_element_type=jnp.float32)
    m = jnp.max(logits, axis=1, keepdims=True)
    ex = jnp.exp(logits - m)
    p = ex / jnp.sum(ex, axis=1, keepdims=True)
    iota = lax.broadcasted_iota(jnp.int32, (BTC, E), 1)
    m1 = jnp.max(p, axis=1, keepdims=True)
    i1 = jnp.min(jnp.where(p == m1, iota, E), axis=1, keepdims=True)
    pm = jnp.where(iota == i1, -jnp.inf, p)
    m2 = jnp.max(pm, axis=1, keepdims=True)
    denom = m1 + m2
    w0 = m1 / denom
    w1 = m2 / denom
    xb = x.astype(jnp.bfloat16)
    h0s = jnp.dot(xb, ws0_s[...], preferred_element_type=jnp.float32)
    h1s = jnp.dot(xb, ws1_s[...], preferred_element_type=jnp.float32)
    hs = (jax.nn.silu(h0s) * h1s).astype(jnp.bfloat16)
    sh = jnp.dot(hs, wso_s[...], preferred_element_type=jnp.float32)
    g = jax.nn.sigmoid(jnp.dot(x, Wg_ref[...],
                               preferred_element_type=jnp.float32))
    out_ref[...] = w0 * op0_ref[...] + w1 * op1_ref[...] + g * sh


def _combine(x, Wr, op0, op1, Ws0, Ws1, Wso, Wg):
    nblk = T // BTC
    return pl.pallas_call(
        _combine_body,
        grid=(nblk,),
        in_specs=[
            pl.BlockSpec((BTC, D), lambda t: (t, 0)),
            pl.BlockSpec((D, E), lambda t: (0, 0)),
            pl.BlockSpec((BTC, D), lambda t: (t, 0)),
            pl.BlockSpec((BTC, D), lambda t: (t, 0)),
            pl.BlockSpec((D, F), lambda t: (0, 0)),
            pl.BlockSpec((D, F), lambda t: (0, 0)),
            pl.BlockSpec((F, D), lambda t: (0, 0)),
            pl.BlockSpec((D, 1), lambda t: (0, 0)),
        ],
        out_specs=pl.BlockSpec((BTC, D), lambda t: (t, 0)),
        out_shape=jax.ShapeDtypeStruct((T, D), jnp.float32),
        scratch_shapes=[
            pltpu.VMEM((D, F), jnp.bfloat16),
            pltpu.VMEM((D, F), jnp.bfloat16),
            pltpu.VMEM((F, D), jnp.bfloat16),
        ],
    )(x, Wr, op0, op1, Ws0, Ws1, Wso, Wg)


@jax.jit
def _moe(x, Wr, W0, W1, Wo, Ws0, Ws1, Wso, Wg):
    mtri = jnp.asarray(_MTRI_NP, jnp.bfloat16)
    slots, meta, loss = _router(x, Wr, mtri)
    xs = _dispatch(x, slots)
    ys = _gmm(meta, xs, W0, W1, Wo)
    op0, op1 = _undispatch(ys, slots)
    out = _combine(x, Wr, op0, op1, Ws0, Ws1, Wso, Wg)
    return out, loss[0, 0]


def kernel(hidden_states, Wr, W_in0, W_in1, W_out, Ws_in0, Ws_in1, Ws_out,
           Wg, deterministic=True):
    b, s, d = hidden_states.shape
    x = hidden_states.reshape(-1, d)
    out, loss = _moe(x, Wr, W_in0, W_in1, W_out, Ws_in0, Ws_in1, Ws_out, Wg)
    return out.reshape(b, s, d), loss


# speedup vs baseline: 4.4170x; 1.0534x over previous
import functools

import numpy as np
import jax
import jax.numpy as jnp
from jax import lax
from jax.experimental import pallas as pl
from jax.experimental.pallas import tpu as pltpu
from jax.experimental.pallas import tpu_sc as plsc

T, D, E, K, F = 2048, 1024, 8, 2, 512
BT = 256
PT = T * K + E * BT
NB = PT // BT
NMETA = 64
BTC = 256
NC, NS = 2, 16
NW = NC * NS
CHUNK = T // NW

TCH = 128
_MTRI_NP = np.triu(np.ones((TCH, TCH), np.float32), 1)


def _router_body(x_ref, Wr_ref, mtri_ref, slots_ref, meta_ref, loss_ref):
    x = x_ref[...]
    logits_t = lax.dot_general(Wr_ref[...], x, (((0,), (1,)), ((), ())),
                               preferred_element_type=jnp.float32)
    m = jnp.max(logits_t, axis=0, keepdims=True)
    ex = jnp.exp(logits_t - m)
    p = ex / jnp.sum(ex, axis=0, keepdims=True)
    eio = lax.broadcasted_iota(jnp.int32, (E, T), 0)
    m1 = jnp.max(p, axis=0, keepdims=True)
    i1 = jnp.min(jnp.where(p == m1, eio, E), axis=0, keepdims=True)
    pm = jnp.where(eio == i1, -jnp.inf, p)
    m2 = jnp.max(pm, axis=0, keepdims=True)
    i2 = jnp.min(jnp.where(pm == m2, eio, E), axis=0, keepdims=True)
    sel = ((eio == i1) | (eio == i2)).astype(jnp.bfloat16)

    mtri = mtri_ref[...]
    parts = []
    carry = jnp.zeros((E, 1), jnp.float32)
    for c in range(T // TCH):
        chunk = sel[:, c * TCH:(c + 1) * TCH]
        within = lax.dot_general(chunk, mtri, (((1,), (0,)), ((), ())),
                                 preferred_element_type=jnp.float32)
        parts.append(within + carry)
        carry = carry + jnp.sum(chunk.astype(jnp.float32), axis=1,
                                keepdims=True)
    rank = jnp.concatenate(parts, axis=1)

    counts = jnp.sum(sel.astype(jnp.float32), axis=1, keepdims=True)
    padded = jnp.ceil(counts / BT) * BT
    eio_r = lax.broadcasted_iota(jnp.int32, (E, E), 0)
    eio_c = lax.broadcasted_iota(jnp.int32, (E, E), 1)
    metri = (eio_c < eio_r).astype(jnp.float32)
    starts = lax.dot_general(metri, padded, (((1,), (0,)), ((), ())),
                             preferred_element_type=jnp.float32)
    slotmat = starts + rank
    slot0 = jnp.sum(jnp.where(eio == i1, slotmat, 0.0), axis=0, keepdims=True)
    slot1 = jnp.sum(jnp.where(eio == i2, slotmat, 0.0), axis=0, keepdims=True)
    slots_ref[...] = jnp.concatenate([slot0, slot1], axis=0).astype(jnp.int32)

    ends = starts + padded
    bio = lax.broadcasted_iota(jnp.int32, (1, NMETA), 1).astype(jnp.float32) * BT
    bexp = jnp.sum((bio >= ends).astype(jnp.int32), axis=0, keepdims=True)
    bexp = jnp.minimum(bexp, E - 1)
    total = jnp.sum(padded, axis=0, keepdims=True)
    bvalid = (bio < total).astype(jnp.int32)
    meta_ref[...] = jnp.concatenate([bexp, bvalid], axis=0)

    psum = jnp.sum(p, axis=1, keepdims=True)
    loss = E * jnp.sum((counts / (T * K)) * (psum / T), keepdims=True)
    loss_ref[...] = loss


def _router(x, Wr, mtri):
    return pl.pallas_call(
        _router_body,
        in_specs=[
            pl.BlockSpec((T, D), lambda: (0, 0)),
            pl.BlockSpec((D, E), lambda: (0, 0)),
            pl.BlockSpec((TCH, TCH), lambda: (0, 0)),
        ],
        out_specs=[
            pl.BlockSpec((2, T), lambda: (0, 0)),
            pl.BlockSpec((2, NMETA), lambda: (0, 0)),
            pl.BlockSpec((1, 1), lambda: (0, 0)),
        ],
        out_shape=[
            jax.ShapeDtypeStruct((2, T), jnp.int32),
            jax.ShapeDtypeStruct((2, NMETA), jnp.int32),
            jax.ShapeDtypeStruct((1, 1), jnp.float32),
        ],
    )(x, Wr, mtri)


_sc_mesh = plsc.VectorSubcoreMesh(core_axis_name="c", subcore_axis_name="s",
                                  num_cores=NC, num_subcores=NS)


@functools.partial(
    pl.kernel,
    out_type=jax.ShapeDtypeStruct((PT, D), jnp.float32),
    mesh=_sc_mesh,
    scratch_types=[
        pltpu.VMEM((CHUNK,), jnp.int32),
        pltpu.VMEM((CHUNK,), jnp.int32),
        pltpu.VMEM((CHUNK, D), jnp.float32),
        pltpu.SemaphoreType.DMA,
        pltpu.SemaphoreType.DMA,
    ],
)
def _dispatch(x_hbm, slots_hbm, xs_hbm, idx0_v, idx1_v, rows_v, sem0, sem1):
    wid = lax.axis_index("s") * NC + lax.axis_index("c")
    base = wid * CHUNK
    a0 = pltpu.async_copy(slots_hbm.at[0, pl.ds(base, CHUNK)], idx0_v, sem0)
    a1 = pltpu.async_copy(slots_hbm.at[1, pl.ds(base, CHUNK)], idx1_v, sem1)
    pltpu.sync_copy(x_hbm.at[pl.ds(base, CHUNK)], rows_v)
    a0.wait()
    a1.wait()
    c0 = pltpu.async_copy(rows_v, xs_hbm.at[idx0_v], sem0)
    c1 = pltpu.async_copy(rows_v, xs_hbm.at[idx1_v], sem1)
    c0.wait()
    c1.wait()


def _gmm_body(meta_ref, xs_ref, W0_ref, W1_ref, Wo_ref, ys_ref):
    b = pl.program_id(0)

    @pl.when(meta_ref[1, b] == 1)
    def _():
        xb = xs_ref[...]
        h0 = jnp.dot(xb, W0_ref[0], preferred_element_type=jnp.float32)
        h1 = jnp.dot(xb, W1_ref[0], preferred_element_type=jnp.float32)
        h = jax.nn.silu(h0) * h1
        ys_ref[...] = jnp.dot(h, Wo_ref[0], preferred_element_type=jnp.float32)


def _gmm(meta, xs, W0, W1, Wo):
    grid_spec = pltpu.PrefetchScalarGridSpec(
        num_scalar_prefetch=1,
        grid=(NB,),
        in_specs=[
            pl.BlockSpec((BT, D), lambda b, meta: (b, 0)),
            pl.BlockSpec((1, D, F), lambda b, meta: (meta[0, b], 0, 0)),
            pl.BlockSpec((1, D, F), lambda b, meta: (meta[0, b], 0, 0)),
            pl.BlockSpec((1, F, D), lambda b, meta: (meta[0, b], 0, 0)),
        ],
        out_specs=pl.BlockSpec((BT, D), lambda b, meta: (b, 0)),
    )
    return pl.pallas_call(
        _gmm_body,
        grid_spec=grid_spec,
        out_shape=jax.ShapeDtypeStruct((PT, D), jnp.float32),
        compiler_params=pltpu.CompilerParams(
            dimension_semantics=("arbitrary",),
        ),
    )(meta, xs, W0, W1, Wo)


@functools.partial(
    pl.kernel,
    out_type=[
        jax.ShapeDtypeStruct((T, D), jnp.float32),
        jax.ShapeDtypeStruct((T, D), jnp.float32),
    ],
    mesh=_sc_mesh,
    scratch_types=[
        pltpu.VMEM((CHUNK,), jnp.int32),
        pltpu.VMEM((CHUNK,), jnp.int32),
        pltpu.VMEM((CHUNK, D), jnp.float32),
        pltpu.SemaphoreType.DMA,
    ],
)
def _undispatch(ys_hbm, slots_hbm, op0_hbm, op1_hbm, idx0_v, idx1_v,
                rows_v, sem):
    wid = lax.axis_index("s") * NC + lax.axis_index("c")
    base = wid * CHUNK
    pltpu.sync_copy(slots_hbm.at[0, pl.ds(base, CHUNK)], idx0_v)
    pltpu.sync_copy(slots_hbm.at[1, pl.ds(base, CHUNK)], idx1_v)
    pltpu.async_copy(ys_hbm.at[idx0_v], rows_v, sem).wait()
    pltpu.sync_copy(rows_v, op0_hbm.at[pl.ds(base, CHUNK)])
    pltpu.async_copy(ys_hbm.at[idx1_v], rows_v, sem).wait()
    pltpu.sync_copy(rows_v, op1_hbm.at[pl.ds(base, CHUNK)])


def _combine_body(x_ref, Wr_ref, op0_ref, op1_ref, Ws0_ref, Ws1_ref,
                  Wso_ref, Wg_ref, out_ref):
    x = x_ref[...]
    logits = jnp.dot(x, Wr_ref[...], preferred_element_type=jnp.float32)
    m = jnp.max(logits, axis=1, keepdims=True)
    ex = jnp.exp(logits - m)
    p = ex / jnp.sum(ex, axis=1, keepdims=True)
    iota = lax.broadcasted_iota(jnp.int32, (BTC, E), 1)
    m1 = jnp.max(p, axis=1, keepdims=True)
    i1 = jnp.min(jnp.where(p == m1, iota, E), axis=1, keepdims=True)
    pm = jnp.where(iota == i1, -jnp.inf, p)
    m2 = jnp.max(pm, axis=1, keepdims=True)
    denom = m1 + m2
    w0 = m1 / denom
    w1 = m2 / denom
    h0s = jnp.dot(x, Ws0_ref[...], preferred_element_type=jnp.float32)
    h1s = jnp.dot(x, Ws1_ref[...], preferred_element_type=jnp.float32)
    sh = jnp.dot(jax.nn.silu(h0s) * h1s, Wso_ref[...],
                 preferred_element_type=jnp.float32)
    g = jax.nn.sigmoid(jnp.dot(x, Wg_ref[...],
                               preferred_element_type=jnp.float32))
    out_ref[...] = w0 * op0_ref[...] + w1 * op1_ref[...] + g * sh


def _combine(x, Wr, op0, op1, Ws0, Ws1, Wso, Wg):
    nblk = T // BTC
    return pl.pallas_call(
        _combine_body,
        grid=(nblk,),
        in_specs=[
            pl.BlockSpec((BTC, D), lambda t: (t, 0)),
            pl.BlockSpec((D, E), lambda t: (0, 0)),
            pl.BlockSpec((BTC, D), lambda t: (t, 0)),
            pl.BlockSpec((BTC, D), lambda t: (t, 0)),
            pl.BlockSpec((D, F), lambda t: (0, 0)),
            pl.BlockSpec((D, F), lambda t: (0, 0)),
            pl.BlockSpec((F, D), lambda t: (0, 0)),
            pl.BlockSpec((D, 1), lambda t: (0, 0)),
        ],
        out_specs=pl.BlockSpec((BTC, D), lambda t: (t, 0)),
        out_shape=jax.ShapeDtypeStruct((T, D), jnp.float32),
    )(x, Wr, op0, op1, Ws0, Ws1, Wso, Wg)


@jax.jit
def _moe(x, Wr, W0, W1, Wo, Ws0, Ws1, Wso, Wg):
    mtri = jnp.asarray(_MTRI_NP, jnp.bfloat16)
    slots, meta, loss = _router(x, Wr, mtri)
    xs = _dispatch(x, slots)
    ys = _gmm(meta, xs, W0, W1, Wo)
    op0, op1 = _undispatch(ys, slots)
    out = _combine(x, Wr, op0, op1, Ws0, Ws1, Wso, Wg)
    return out, loss[0, 0]


def kernel(hidden_states, Wr, W_in0, W_in1, W_out, Ws_in0, Ws_in1, Ws_out,
           Wg, deterministic=True):
    b, s, d = hidden_states.shape
    x = hidden_states.reshape(-1, d)
    out, loss = _moe(x, Wr, W_in0, W_in1, W_out, Ws_in0, Ws_in1, Ws_out, Wg)
    return out.reshape(b, s, d), loss


# speedup vs baseline: 4.4393x; 1.0050x over previous
import functools

import numpy as np
import jax
import jax.numpy as jnp
from jax import lax
from jax.experimental import pallas as pl
from jax.experimental.pallas import tpu as pltpu
from jax.experimental.pallas import tpu_sc as plsc

T, D, E, K, F = 2048, 1024, 8, 2, 512
BT = 256
PT = T * K + E * BT
NB = PT // BT
NMETA = 64
BTC = 256
NC, NS = 2, 16
NW = NC * NS
CHUNK = T // NW

TCH = 128
_MTRI_NP = np.triu(np.ones((TCH, TCH), np.float32), 1)


def _router_body(x_ref, Wr_ref, mtri_ref, slots_ref, meta_ref, loss_ref):
    x = x_ref[...]
    logits_t = lax.dot_general(Wr_ref[...], x, (((0,), (1,)), ((), ())),
                               preferred_element_type=jnp.float32)
    m = jnp.max(logits_t, axis=0, keepdims=True)
    ex = jnp.exp(logits_t - m)
    p = ex / jnp.sum(ex, axis=0, keepdims=True)
    eio = lax.broadcasted_iota(jnp.int32, (E, T), 0)
    m1 = jnp.max(p, axis=0, keepdims=True)
    i1 = jnp.min(jnp.where(p == m1, eio, E), axis=0, keepdims=True)
    pm = jnp.where(eio == i1, -jnp.inf, p)
    m2 = jnp.max(pm, axis=0, keepdims=True)
    i2 = jnp.min(jnp.where(pm == m2, eio, E), axis=0, keepdims=True)
    sel = ((eio == i1) | (eio == i2)).astype(jnp.bfloat16)

    mtri = mtri_ref[...]
    parts = []
    carry = jnp.zeros((E, 1), jnp.float32)
    for c in range(T // TCH):
        chunk = sel[:, c * TCH:(c + 1) * TCH]
        within = lax.dot_general(chunk, mtri, (((1,), (0,)), ((), ())),
                                 preferred_element_type=jnp.float32)
        parts.append(within + carry)
        carry = carry + jnp.sum(chunk.astype(jnp.float32), axis=1,
                                keepdims=True)
    rank = jnp.concatenate(parts, axis=1)

    counts = jnp.sum(sel.astype(jnp.float32), axis=1, keepdims=True)
    padded = jnp.ceil(counts / BT) * BT
    eio_r = lax.broadcasted_iota(jnp.int32, (E, E), 0)
    eio_c = lax.broadcasted_iota(jnp.int32, (E, E), 1)
    metri = (eio_c < eio_r).astype(jnp.float32)
    starts = lax.dot_general(metri, padded, (((1,), (0,)), ((), ())),
                             preferred_element_type=jnp.float32)
    slotmat = starts + rank
    slot0 = jnp.sum(jnp.where(eio == i1, slotmat, 0.0), axis=0, keepdims=True)
    slot1 = jnp.sum(jnp.where(eio == i2, slotmat, 0.0), axis=0, keepdims=True)
    slots_ref[...] = jnp.concatenate([slot0, slot1], axis=0).astype(jnp.int32)

    ends = starts + padded
    bio = lax.broadcasted_iota(jnp.int32, (1, NMETA), 1).astype(jnp.float32) * BT
    bexp = jnp.sum((bio >= ends).astype(jnp.int32), axis=0, keepdims=True)
    bexp = jnp.minimum(bexp, E - 1)
    total = jnp.sum(padded, axis=0, keepdims=True)
    bvalid = (bio < total).astype(jnp.int32)
    meta_ref[...] = jnp.concatenate([bexp, bvalid], axis=0)

    psum = jnp.sum(p, axis=1, keepdims=True)
    loss = E * jnp.sum((counts / (T * K)) * (psum / T), keepdims=True)
    loss_ref[...] = loss


def _router(x, Wr, mtri):
    return pl.pallas_call(
        _router_body,
        in_specs=[
            pl.BlockSpec((T, D), lambda: (0, 0)),
            pl.BlockSpec((D, E), lambda: (0, 0)),
            pl.BlockSpec((TCH, TCH), lambda: (0, 0)),
        ],
        out_specs=[
            pl.BlockSpec((2, T), lambda: (0, 0)),
            pl.BlockSpec((2, NMETA), lambda: (0, 0)),
            pl.BlockSpec((1, 1), lambda: (0, 0)),
        ],
        out_shape=[
            jax.ShapeDtypeStruct((2, T), jnp.int32),
            jax.ShapeDtypeStruct((2, NMETA), jnp.int32),
            jax.ShapeDtypeStruct((1, 1), jnp.float32),
        ],
    )(x, Wr, mtri)


_sc_mesh = plsc.VectorSubcoreMesh(core_axis_name="c", subcore_axis_name="s",
                                  num_cores=NC, num_subcores=NS)


@functools.partial(
    pl.kernel,
    out_type=jax.ShapeDtypeStruct((PT, D), jnp.float32),
    mesh=_sc_mesh,
    scratch_types=[
        pltpu.VMEM((CHUNK,), jnp.int32),
        pltpu.VMEM((CHUNK,), jnp.int32),
        pltpu.VMEM((CHUNK, D), jnp.float32),
        pltpu.SemaphoreType.DMA,
        pltpu.SemaphoreType.DMA,
    ],
)
def _dispatch(x_hbm, slots_hbm, xs_hbm, idx0_v, idx1_v, rows_v, sem0, sem1):
    wid = lax.axis_index("s") * NC + lax.axis_index("c")
    base = wid * CHUNK
    a0 = pltpu.async_copy(slots_hbm.at[0, pl.ds(base, CHUNK)], idx0_v, sem0)
    a1 = pltpu.async_copy(slots_hbm.at[1, pl.ds(base, CHUNK)], idx1_v, sem1)
    pltpu.sync_copy(x_hbm.at[pl.ds(base, CHUNK)], rows_v)
    a0.wait()
    a1.wait()
    c0 = pltpu.async_copy(rows_v, xs_hbm.at[idx0_v], sem0)
    c1 = pltpu.async_copy(rows_v, xs_hbm.at[idx1_v], sem1)
    c0.wait()
    c1.wait()


def _gmm_body(meta_ref, xs_ref, W0_ref, W1_ref, Wo_ref, ys_ref):
    b = pl.program_id(0)

    @pl.when(meta_ref[1, b] == 1)
    def _():
        xb = xs_ref[...]
        h0 = jnp.dot(xb, W0_ref[0], preferred_element_type=jnp.float32)
        h1 = jnp.dot(xb, W1_ref[0], preferred_element_type=jnp.float32)
        h = jax.nn.silu(h0) * h1
        ys_ref[...] = jnp.dot(h, Wo_ref[0], preferred_element_type=jnp.float32)


def _gmm(meta, xs, W0, W1, Wo):
    grid_spec = pltpu.PrefetchScalarGridSpec(
        num_scalar_prefetch=1,
        grid=(NB,),
        in_specs=[
            pl.BlockSpec((BT, D), lambda b, meta: (b, 0)),
            pl.BlockSpec((1, D, F), lambda b, meta: (meta[0, b], 0, 0)),
            pl.BlockSpec((1, D, F), lambda b, meta: (meta[0, b], 0, 0)),
            pl.BlockSpec((1, F, D), lambda b, meta: (meta[0, b], 0, 0)),
        ],
        out_specs=pl.BlockSpec((BT, D), lambda b, meta: (b, 0)),
    )
    return pl.pallas_call(
        _gmm_body,
        grid_spec=grid_spec,
        out_shape=jax.ShapeDtypeStruct((PT, D), jnp.float32),
        compiler_params=pltpu.CompilerParams(
            dimension_semantics=("arbitrary",),
        ),
    )(meta, xs, W0, W1, Wo)


@functools.partial(
    pl.kernel,
    out_type=[
        jax.ShapeDtypeStruct((T, D), jnp.float32),
        jax.ShapeDtypeStruct((T, D), jnp.float32),
    ],
    mesh=_sc_mesh,
    scratch_types=[
        pltpu.VMEM((CHUNK,), jnp.int32),
        pltpu.VMEM((CHUNK,), jnp.int32),
        pltpu.VMEM((CHUNK, D), jnp.float32),
        pltpu.SemaphoreType.DMA,
        pltpu.SemaphoreType.DMA,
    ],
)
def _undispatch(ys_hbm, slots_hbm, op0_hbm, op1_hbm, idx0_v, idx1_v,
                rows_v, sem, sem2):
    wid = lax.axis_index("s") * NC + lax.axis_index("c")
    base = wid * CHUNK
    a0 = pltpu.async_copy(slots_hbm.at[0, pl.ds(base, CHUNK)], idx0_v, sem)
    a1 = pltpu.async_copy(slots_hbm.at[1, pl.ds(base, CHUNK)], idx1_v, sem2)
    a0.wait()
    a1.wait()
    pltpu.async_copy(ys_hbm.at[idx0_v], rows_v, sem).wait()
    pltpu.sync_copy(rows_v, op0_hbm.at[pl.ds(base, CHUNK)])
    pltpu.async_copy(ys_hbm.at[idx1_v], rows_v, sem).wait()
    pltpu.sync_copy(rows_v, op1_hbm.at[pl.ds(base, CHUNK)])


def _combine_body(x_ref, Wr_ref, op0_ref, op1_ref, Ws0_ref, Ws1_ref,
                  Wso_ref, Wg_ref, out_ref):
    x = x_ref[...]
    logits = jnp.dot(x, Wr_ref[...], preferred_element_type=jnp.float32)
    m = jnp.max(logits, axis=1, keepdims=True)
    ex = jnp.exp(logits - m)
    p = ex / jnp.sum(ex, axis=1, keepdims=True)
    iota = lax.broadcasted_iota(jnp.int32, (BTC, E), 1)
    m1 = jnp.max(p, axis=1, keepdims=True)
    i1 = jnp.min(jnp.where(p == m1, iota, E), axis=1, keepdims=True)
    pm = jnp.where(iota == i1, -jnp.inf, p)
    m2 = jnp.max(pm, axis=1, keepdims=True)
    denom = m1 + m2
    w0 = m1 / denom
    w1 = m2 / denom
    h0s = jnp.dot(x, Ws0_ref[...], preferred_element_type=jnp.float32)
    h1s = jnp.dot(x, Ws1_ref[...], preferred_element_type=jnp.float32)
    sh = jnp.dot(jax.nn.silu(h0s) * h1s, Wso_ref[...],
                 preferred_element_type=jnp.float32)
    g = jax.nn.sigmoid(jnp.dot(x, Wg_ref[...],
                               preferred_element_type=jnp.float32))
    out_ref[...] = w0 * op0_ref[...] + w1 * op1_ref[...] + g * sh


def _combine(x, Wr, op0, op1, Ws0, Ws1, Wso, Wg):
    nblk = T // BTC
    return pl.pallas_call(
        _combine_body,
        grid=(nblk,),
        in_specs=[
            pl.BlockSpec((BTC, D), lambda t: (t, 0)),
            pl.BlockSpec((D, E), lambda t: (0, 0)),
            pl.BlockSpec((BTC, D), lambda t: (t, 0)),
            pl.BlockSpec((BTC, D), lambda t: (t, 0)),
            pl.BlockSpec((D, F), lambda t: (0, 0)),
            pl.BlockSpec((D, F), lambda t: (0, 0)),
            pl.BlockSpec((F, D), lambda t: (0, 0)),
            pl.BlockSpec((D, 1), lambda t: (0, 0)),
        ],
        out_specs=pl.BlockSpec((BTC, D), lambda t: (t, 0)),
        out_shape=jax.ShapeDtypeStruct((T, D), jnp.float32),
    )(x, Wr, op0, op1, Ws0, Ws1, Wso, Wg)


@jax.jit
def _moe(x, Wr, W0, W1, Wo, Ws0, Ws1, Wso, Wg):
    mtri = jnp.asarray(_MTRI_NP, jnp.bfloat16)
    slots, meta, loss = _router(x, Wr, mtri)
    xs = _dispatch(x, slots)
    ys = _gmm(meta, xs, W0, W1, Wo)
    op0, op1 = _undispatch(ys, slots)
    out = _combine(x, Wr, op0, op1, Ws0, Ws1, Wso, Wg)
    return out, loss[0, 0]


def kernel(hidden_states, Wr, W_in0, W_in1, W_out, Ws_in0, Ws_in1, Ws_out,
           Wg, deterministic=True):
    b, s, d = hidden_states.shape
    x = hidden_states.reshape(-1, d)
    out, loss = _moe(x, Wr, W_in0, W_in1, W_out, Ws_in0, Ws_in1, Ws_out, Wg)
    return out.reshape(b, s, d), loss
